# baseline (device time: 7802 ns/iter reference)
import jax
import jax.numpy as jnp
from jax import lax
from jax.experimental import pallas as pl
from jax.experimental.pallas import tpu as pltpu

N_GLOBAL = 1536
K = 4
LANES = 128


def kernel(x):
    m, n = x.shape
    blk = m // K
    pk = blk // LANES
    rows = m // LANES

    def body(x_ref, out_ref, send_ref, recv_ref, send_sems, recv_sems):
        k = pl.program_id(0)
        my_x = lax.axis_index("x")
        my_y = lax.axis_index("y")
        peer = (my_x, 1 - my_y)

        def chunk_rdma(j, sem_idx):
            return pltpu.make_async_remote_copy(
                src_ref=send_ref.at[pl.ds(j * pk, pk)],
                dst_ref=recv_ref.at[pl.ds(j * pk, pk)],
                send_sem=send_sems.at[sem_idx],
                recv_sem=recv_sems.at[sem_idx],
                device_id=peer,
                device_id_type=pl.DeviceIdType.MESH,
            )

        @pl.when(k == 0)
        def _():
            barrier_sem = pltpu.get_barrier_semaphore()
            pl.semaphore_signal(
                barrier_sem, inc=1, device_id=peer,
                device_id_type=pl.DeviceIdType.MESH,
            )
            pl.semaphore_wait(barrier_sem, 1)

        partial = jnp.sum(x_ref[:, :], axis=1)
        send_ref[pl.ds(k * pk, pk), :] = jnp.reshape(partial, (pk, LANES))
        chunk_rdma(k, k).start()

        @pl.when(k == K - 1)
        def _():
            for j in range(K):
                chunk_rdma(j, j).wait()

            total = (send_ref[:, :] + recv_ref[:, :]) * (1.0 / N_GLOBAL)
            eye = jnp.eye(LANES, dtype=jnp.float32)
            cols = jax.lax.dot_general(
                eye, total, (((1,), (1,)), ((), ()))
            )
            for j in range(rows):
                out_ref[pl.ds(j * LANES, LANES), :] = cols[:, j : j + 1]

    return pl.pallas_call(
        body,
        grid=(K,),
        out_shape=jax.ShapeDtypeStruct((m, 1), jnp.float32),
        in_specs=[
            pl.BlockSpec((blk, n), lambda k: (k, 0), memory_space=pltpu.VMEM)
        ],
        out_specs=pl.BlockSpec(
            (m, 1), lambda k: (0, 0), memory_space=pltpu.VMEM
        ),
        scratch_shapes=[
            pltpu.VMEM((rows, LANES), jnp.float32),
            pltpu.VMEM((rows, LANES), jnp.float32),
            pltpu.SemaphoreType.DMA((K,)),
            pltpu.SemaphoreType.DMA((K,)),
        ],
        compiler_params=pltpu.CompilerParams(collective_id=0),
    )(x)
